# baseline (speedup 1.0000x reference)
import jax
import jax.numpy as jnp
from jax.experimental import pallas as pl
from jax.experimental.pallas import tpu as pltpu

_TEMP_INV = 10.0
_N = 2048
_R = 8
_BI = 256
_NK = _N // _BI


def _body(p_ref, t_ref, out_ref, acc_ref, tp_ref, tt_ref):
    r = pl.program_id(0)
    i_idx = pl.program_id(1)

    @pl.when(jnp.logical_and(r == 0, i_idx == 0))
    def _():
        acc_ref[3] = 0.0

    @pl.when(i_idx == 0)
    def _():
        acc_ref[0] = 0.0
        acc_ref[1] = 0.0
        acc_ref[2] = 0.0
        tp_ref[0, :] = jnp.zeros((_N,), jnp.float32)
        tt_ref[0, :] = jnp.zeros((_N,), jnp.float32)

    ioff = pl.multiple_of(i_idx * _BI, _BI)
    pi = p_ref[0, 0, pl.ds(ioff, _BI)].reshape(_BI, 1)
    ti = t_ref[0, 0, pl.ds(ioff, _BI)].reshape(_BI, 1)

    def jbody(j_idx, carry):
        rs_p, rs_t = carry
        joff = pl.multiple_of(j_idx * _BI, _BI)
        pj = p_ref[0, 0, pl.ds(joff, _BI)].reshape(1, _BI)
        tj = t_ref[0, 0, pl.ds(joff, _BI)].reshape(1, _BI)

        bp = jnp.tanh((pi - pj) * (0.5 * _TEMP_INV))
        bt = jnp.tanh((ti - tj) * (0.5 * _TEMP_INV))

        @pl.when(j_idx > i_idx)
        def _():
            tp_ref[0, pl.ds(joff, _BI)] -= jnp.sum(bp, axis=0)
            tt_ref[0, pl.ds(joff, _BI)] -= jnp.sum(bt, axis=0)

        return (rs_p + jnp.sum(bp, axis=1), rs_t + jnp.sum(bt, axis=1))

    zeros = jnp.zeros((_BI,), jnp.float32)
    rs_p, rs_t = jax.lax.fori_loop(i_idx, _NK, jbody, (zeros, zeros))

    xb = 0.5 * (tp_ref[0, pl.ds(ioff, _BI)] + rs_p)
    yb = 0.5 * (tt_ref[0, pl.ds(ioff, _BI)] + rs_t)
    acc_ref[0] += jnp.sum(xb * yb)
    acc_ref[1] += jnp.sum(xb * xb)
    acc_ref[2] += jnp.sum(yb * yb)

    @pl.when(i_idx == _NK - 1)
    def _():
        sxy = acc_ref[0] / _N
        sxx = acc_ref[1] / _N
        syy = acc_ref[2] / _N
        vx = jnp.sqrt(sxx + 1e-8)
        vy = jnp.sqrt(syy + 1e-8)
        rho = sxy / (vx * vy + 1e-8)
        acc_ref[3] += (1.0 - rho) / _R

    @pl.when(jnp.logical_and(r == _R - 1, i_idx == _NK - 1))
    def _():
        out_ref[0, 0] = acc_ref[3]


def kernel(preds, targets):
    p3 = preds.reshape(_R, 1, _N)
    t3 = targets.reshape(_R, 1, _N)
    out = pl.pallas_call(
        _body,
        grid=(_R, _NK),
        in_specs=[
            pl.BlockSpec((1, 1, _N), lambda r, i: (r, 0, 0)),
            pl.BlockSpec((1, 1, _N), lambda r, i: (r, 0, 0)),
        ],
        out_specs=pl.BlockSpec(memory_space=pltpu.SMEM),
        out_shape=jax.ShapeDtypeStruct((1, 1), jnp.float32),
        scratch_shapes=[
            pltpu.SMEM((4,), jnp.float32),
            pltpu.VMEM((1, _N), jnp.float32),
            pltpu.VMEM((1, _N), jnp.float32),
        ],
    )(p3, t3)
    return out[0, 0]


# speedup vs baseline: 2.5359x; 2.5359x over previous
import jax
import jax.numpy as jnp
from jax.experimental import pallas as pl
from jax.experimental.pallas import tpu as pltpu

_TEMP_INV = 10.0
_N = 2048
_R = 8
_BI = 256
_NK = _N // _BI


def _body(p_ref, t_ref, out_ref, acc_ref, tp_ref, tt_ref):
    r = pl.program_id(0)

    @pl.when(r == 0)
    def _():
        acc_ref[0] = 0.0

    tp_ref[0, :] = jnp.zeros((_N,), jnp.float32)
    tt_ref[0, :] = jnp.zeros((_N,), jnp.float32)

    ap = p_ref[0, 0, :] * (0.5 * _TEMP_INV)
    at = t_ref[0, 0, :] * (0.5 * _TEMP_INV)

    sxy = 0.0
    sxx = 0.0
    syy = 0.0
    for i in range(_NK):
        lo = i * _BI
        hi = (i + 1) * _BI
        w = _N - lo

        pi = ap[lo:hi].reshape(_BI, 1)
        ps = ap[lo:].reshape(1, w)
        bp = jnp.tanh(pi - ps)
        ti = at[lo:hi].reshape(_BI, 1)
        ts = at[lo:].reshape(1, w)
        bt = jnp.tanh(ti - ts)

        bp_h = bp.astype(jnp.bfloat16)
        bt_h = bt.astype(jnp.bfloat16)
        ones_col = jnp.ones((w, 1), jnp.bfloat16)
        ones_row = jnp.ones((1, _BI), jnp.bfloat16)
        dims = (((1,), (0,)), ((), ()))
        rs_p = jax.lax.dot_general(bp_h, ones_col, dims,
                                   preferred_element_type=jnp.float32)
        rs_t = jax.lax.dot_general(bt_h, ones_col, dims,
                                   preferred_element_type=jnp.float32)
        cs_p = jax.lax.dot_general(ones_row, bp_h, dims,
                                   preferred_element_type=jnp.float32)
        cs_t = jax.lax.dot_general(ones_row, bt_h, dims,
                                   preferred_element_type=jnp.float32)

        xb = 0.5 * (tp_ref[0, lo:hi] + rs_p.reshape(_BI))
        yb = 0.5 * (tt_ref[0, lo:hi] + rs_t.reshape(_BI))
        if i < _NK - 1:
            tp_ref[0, hi:] -= cs_p[0, _BI:]
            tt_ref[0, hi:] -= cs_t[0, _BI:]

        sxy += jnp.sum(xb * yb)
        sxx += jnp.sum(xb * xb)
        syy += jnp.sum(yb * yb)

    vx = jnp.sqrt(sxx / _N + 1e-8)
    vy = jnp.sqrt(syy / _N + 1e-8)
    rho = (sxy / _N) / (vx * vy + 1e-8)
    acc_ref[0] += (1.0 - rho) / _R

    @pl.when(r == _R - 1)
    def _():
        out_ref[0, 0] = acc_ref[0]


def kernel(preds, targets):
    p3 = preds.reshape(_R, 1, _N)
    t3 = targets.reshape(_R, 1, _N)
    out = pl.pallas_call(
        _body,
        grid=(_R,),
        in_specs=[
            pl.BlockSpec((1, 1, _N), lambda r: (r, 0, 0)),
            pl.BlockSpec((1, 1, _N), lambda r: (r, 0, 0)),
        ],
        out_specs=pl.BlockSpec(memory_space=pltpu.SMEM),
        out_shape=jax.ShapeDtypeStruct((1, 1), jnp.float32),
        scratch_shapes=[
            pltpu.SMEM((1,), jnp.float32),
            pltpu.VMEM((1, _N), jnp.float32),
            pltpu.VMEM((1, _N), jnp.float32),
        ],
    )(p3, t3)
    return out[0, 0]
